# Initial kernel scaffold; baseline (speedup 1.0000x reference)
import jax
import jax.numpy as jnp
from jax.experimental import pallas as pl


def kernel(seq_ids, seq_len, table):
    raise NotImplementedError("write your pallas kernel here")


# speedup vs baseline: 1.1708x; 1.1708x over previous
import functools

import jax
import jax.numpy as jnp
from jax import lax
from jax.experimental import pallas as pl
from jax.experimental.pallas import tpu as pltpu
from jax.experimental.pallas import tpu_sc as plsc

VOCAB = 30522
D = 768
B = 16
L = 512

NC = 2
NS = 16
NW = NC * NS
TOK = B * L
CH = TOK // NW
P = 64
NP = CH // P
DV = D // 16


def _body(ids_hbm, len_hbm, table_hbm, out_hbm, idx_p, slv, buf, sem):
    wid = lax.axis_index("s") * NC + lax.axis_index("c")
    base_tok = wid * CH
    b = wid // (L // CH)
    l_start = (wid % (L // CH)) * CH

    pltpu.sync_copy(len_hbm, slv)

    lane = lax.broadcasted_iota(jnp.int32, (16,), 0)
    sl = jnp.max(jnp.where(lane == b, slv[...], 0))
    nv = lax.max(lax.min(sl - l_start, CH), 0)

    zeros16 = jnp.zeros((16,), jnp.float32)

    for i in range(NP):
        lo = lax.max(lax.min(nv - i * P, P), 0)
        row0 = base_tok + i * P

        pltpu.sync_copy(ids_hbm.at[pl.ds(row0, P)], idx_p)
        pltpu.async_copy(table_hbm.at[idx_p], buf, sem).wait()

        def zo(r, _):
            for c in range(DV):
                buf[r, pl.ds(c * 16, 16)] = zeros16
            return 0

        lax.fori_loop(lo, P, zo, 0)
        pltpu.sync_copy(buf, out_hbm.at[pl.ds(row0, P)])


@functools.partial(jax.jit, static_argnames=())
def kernel(seq_ids, seq_len, table):
    ids = seq_ids.reshape(TOK).astype(jnp.int32)
    slen = seq_len.astype(jnp.int32)
    tab = table.astype(jnp.float32)

    run = pl.kernel(
        _body,
        out_type=jax.ShapeDtypeStruct((TOK, D), jnp.float32),
        mesh=plsc.VectorSubcoreMesh(core_axis_name="c", subcore_axis_name="s"),
        compiler_params=pltpu.CompilerParams(needs_layout_passes=False),
        scratch_types=[
            pltpu.VMEM((P,), jnp.int32),
            pltpu.VMEM((16,), jnp.int32),
            pltpu.VMEM((P, D), jnp.float32),
            pltpu.SemaphoreType.DMA,
        ],
    )
    out = run(ids, slen, tab)
    return out.reshape(B, L, D)


# speedup vs baseline: 1.3702x; 1.1704x over previous
import functools

import jax
import jax.numpy as jnp
from jax import lax
from jax.experimental import pallas as pl
from jax.experimental.pallas import tpu as pltpu
from jax.experimental.pallas import tpu_sc as plsc

VOCAB = 30522
D = 768
B = 16
L = 512

NC = 2
NS = 16
NW = NC * NS
TOK = B * L
CH = TOK // NW
P = 64
NP = CH // P
DV = D // 16


def _body(ids_hbm, len_hbm, table_hbm, out_hbm,
          idx0, idx1, idx2, idx3, slv, bufA, bufB,
          isem, g0, g1, s0, s1):
    wid = lax.axis_index("s") * NC + lax.axis_index("c")
    base_tok = wid * CH
    b = wid // (L // CH)
    l_start = (wid % (L // CH)) * CH

    idxs = (idx0, idx1, idx2, idx3)
    bufs = (bufA, bufB)
    gsems = (g0, g1)
    ssems = (s0, s1)

    for i in range(NP):
        pltpu.make_async_copy(
            ids_hbm.at[pl.ds(base_tok + i * P, P)], idxs[i], isem
        ).start()
    pltpu.sync_copy(len_hbm, slv)
    for i in range(NP):
        pltpu.make_async_copy(
            ids_hbm.at[pl.ds(base_tok + i * P, P)], idxs[i], isem
        ).wait()

    lane = lax.broadcasted_iota(jnp.int32, (16,), 0)
    sl = jnp.max(jnp.where(lane == b, slv[...], 0))
    nv = lax.max(lax.min(sl - l_start, CH), 0)

    zeros16 = jnp.zeros((16,), jnp.float32)

    pltpu.make_async_copy(table_hbm.at[idxs[0]], bufs[0], gsems[0]).start()
    pltpu.make_async_copy(table_hbm.at[idxs[1]], bufs[1], gsems[1]).start()

    for i in range(NP):
        s = i & 1
        buf = bufs[s]
        lo = lax.max(lax.min(nv - i * P, P), 0)
        row0 = base_tok + i * P

        pltpu.make_async_copy(table_hbm.at[idxs[i]], buf, gsems[s]).wait()

        def zo(r, _):
            for c in range(DV):
                buf[r, pl.ds(c * 16, 16)] = zeros16
            return 0

        lax.fori_loop(lo, P, zo, 0)

        pltpu.make_async_copy(buf, out_hbm.at[pl.ds(row0, P)], ssems[s]).start()
        if i + 2 < NP:
            pltpu.make_async_copy(
                buf, out_hbm.at[pl.ds(row0, P)], ssems[s]).wait()
            pltpu.make_async_copy(
                table_hbm.at[idxs[i + 2]], buf, gsems[s]).start()

    for i in (NP - 2, NP - 1):
        s = i & 1
        pltpu.make_async_copy(
            bufs[s], out_hbm.at[pl.ds(base_tok + i * P, P)], ssems[s]).wait()


@functools.partial(jax.jit, static_argnames=())
def kernel(seq_ids, seq_len, table):
    ids = seq_ids.reshape(TOK).astype(jnp.int32)
    slen = seq_len.astype(jnp.int32)
    tab = table.astype(jnp.float32)

    run = pl.kernel(
        _body,
        out_type=jax.ShapeDtypeStruct((TOK, D), jnp.float32),
        mesh=plsc.VectorSubcoreMesh(core_axis_name="c", subcore_axis_name="s"),
        compiler_params=pltpu.CompilerParams(needs_layout_passes=False),
        scratch_types=[
            pltpu.VMEM((P,), jnp.int32),
            pltpu.VMEM((P,), jnp.int32),
            pltpu.VMEM((P,), jnp.int32),
            pltpu.VMEM((P,), jnp.int32),
            pltpu.VMEM((16,), jnp.int32),
            pltpu.VMEM((P, D), jnp.float32),
            pltpu.VMEM((P, D), jnp.float32),
            pltpu.SemaphoreType.DMA,
            pltpu.SemaphoreType.DMA,
            pltpu.SemaphoreType.DMA,
            pltpu.SemaphoreType.DMA,
            pltpu.SemaphoreType.DMA,
        ],
    )
    out = run(ids, slen, tab)
    return out.reshape(B, L, D)


# speedup vs baseline: 1.3724x; 1.0016x over previous
import functools

import jax
import jax.numpy as jnp
from jax import lax
from jax.experimental import pallas as pl
from jax.experimental.pallas import tpu as pltpu
from jax.experimental.pallas import tpu_sc as plsc

VOCAB = 30522
D = 768
B = 16
L = 512

NC = 2
NS = 16
NW = NC * NS
TOK = B * L
CH = TOK // NW
P = 32
NP = CH // P
NBUF = 4
DV = D // 16


def _body(ids_hbm, len_hbm, table_hbm, out_hbm,
          idx_refs, slv, buf_refs, isem, gsems, ssems):
    wid = lax.axis_index("s") * NC + lax.axis_index("c")
    base_tok = wid * CH
    b = wid // (L // CH)
    l_start = (wid % (L // CH)) * CH

    for i in range(NP):
        pltpu.make_async_copy(
            ids_hbm.at[pl.ds(base_tok + i * P, P)], idx_refs[i], isem
        ).start()
    pltpu.sync_copy(len_hbm, slv)
    for i in range(NP):
        pltpu.make_async_copy(
            ids_hbm.at[pl.ds(base_tok + i * P, P)], idx_refs[i], isem
        ).wait()

    lane = lax.broadcasted_iota(jnp.int32, (16,), 0)
    sl = jnp.max(jnp.where(lane == b, slv[...], 0))
    nv = lax.max(lax.min(sl - l_start, CH), 0)

    zeros16 = jnp.zeros((16,), jnp.float32)

    def gather(i, s):
        pltpu.make_async_copy(
            table_hbm.at[idx_refs[i]], buf_refs[s], gsems[s]).start()

    def scat(i, s):
        return pltpu.make_async_copy(
            buf_refs[s], out_hbm.at[pl.ds(base_tok + i * P, P)], ssems[s])

    gather(0, 0)
    gather(1, 1)

    for j in range(NP):
        s = j % NBUF
        buf = buf_refs[s]
        lo = lax.max(lax.min(nv - j * P, P), 0)

        pltpu.make_async_copy(
            table_hbm.at[idx_refs[j]], buf, gsems[s]).wait()

        def zo(r, _):
            for c in range(DV):
                buf[r, pl.ds(c * 16, 16)] = zeros16
            return 0

        lax.fori_loop(lo, P, zo, 0)

        scat(j, s).start()

        if j + 2 < NP:
            s2 = (j + 2) % NBUF
            if j - 2 >= 0:
                scat(j - 2, s2).wait()
            gather(j + 2, s2)

    for j in range(max(NP - NBUF, 0), NP):
        scat(j, j % NBUF).wait()


@functools.partial(jax.jit, static_argnames=())
def kernel(seq_ids, seq_len, table):
    ids = seq_ids.reshape(TOK).astype(jnp.int32)
    slen = seq_len.astype(jnp.int32)
    tab = table.astype(jnp.float32)

    def body(ids_hbm, len_hbm, table_hbm, out_hbm, *rest):
        idx_refs = rest[:NP]
        slv = rest[NP]
        buf_refs = rest[NP + 1:NP + 1 + NBUF]
        isem = rest[NP + 1 + NBUF]
        gsems = rest[NP + 2 + NBUF:NP + 2 + NBUF + NBUF]
        ssems = rest[NP + 2 + 2 * NBUF:]
        _body(ids_hbm, len_hbm, table_hbm, out_hbm,
              idx_refs, slv, buf_refs, isem, gsems, ssems)

    run = pl.kernel(
        body,
        out_type=jax.ShapeDtypeStruct((TOK, D), jnp.float32),
        mesh=plsc.VectorSubcoreMesh(core_axis_name="c", subcore_axis_name="s"),
        compiler_params=pltpu.CompilerParams(needs_layout_passes=False),
        scratch_types=(
            [pltpu.VMEM((P,), jnp.int32) for _ in range(NP)]
            + [pltpu.VMEM((16,), jnp.int32)]
            + [pltpu.VMEM((P, D), jnp.float32) for _ in range(NBUF)]
            + [pltpu.SemaphoreType.DMA]
            + [pltpu.SemaphoreType.DMA for _ in range(2 * NBUF)]
        ),
    )
    out = run(ids, slen, tab)
    return out.reshape(B, L, D)
